# baseline (speedup 1.0000x reference)
import functools

import jax
import jax.numpy as jnp
from jax import lax
from jax.experimental import pallas as pl
from jax.experimental.pallas import tpu as pltpu
from jax.experimental.pallas import tpu_sc as plsc


def _sc_aggregate(x, edge_index, n_nodes, n_edges, d):
    info = plsc.get_sparse_core_info()
    nc, ns = info.num_cores, info.num_subcores
    nw = nc * ns

    epw = n_edges // nw
    chunk = 40
    n_chunks = epw // chunk
    nbuf = 5
    grp = 25
    n_grps = n_chunks // grp
    n_rounds = grp // nbuf
    assert epw % chunk == 0 and n_edges % nw == 0
    assert n_chunks % grp == 0 and grp % nbuf == 0 and n_rounds >= 2
    rows_blk = (n_nodes // ns) // 8 * 8
    tail = n_nodes - ns * rows_blk
    zblk = 16
    assert rows_blk % zblk == 0 and zblk % 8 == 0 and tail % 8 == 0
    assert tail <= zblk

    mesh = plsc.VectorSubcoreMesh(core_axis_name="c", subcore_axis_name="s")

    @functools.partial(
        pl.kernel,
        mesh=mesh,
        out_type=jax.ShapeDtypeStruct((nc, n_nodes, d), jnp.float32),
        scratch_types=(
            [
                pltpu.VMEM((grp, chunk), jnp.int32),
                pltpu.VMEM((grp, chunk), jnp.int32),
                pltpu.VMEM((grp, chunk), jnp.int32),
                pltpu.VMEM((grp, chunk), jnp.int32),
            ]
            + [pltpu.VMEM((chunk, d), jnp.float32) for _ in range(nbuf)]
            + [
                pltpu.VMEM_SHARED((n_nodes, d), jnp.float32),
                pltpu.VMEM((zblk, d), jnp.float32),
            ]
            + [pltpu.SemaphoreType.DMA for _ in range(2 * nbuf + 1)]
        ),
    )
    def agg_kernel(x_hbm, e_hbm, out_hbm, src0_v, dst0_v, src1_v, dst1_v,
                   *rest):
        srcs = (src0_v, src1_v)
        dsts = (dst0_v, dst1_v)
        bufs = rest[:nbuf]
        acc_sh = rest[nbuf]
        zero_v = rest[nbuf + 1]
        gsem = rest[nbuf + 2:2 * nbuf + 2]
        ssem = rest[2 * nbuf + 2:3 * nbuf + 2]
        zsem = rest[3 * nbuf + 2]
        c = lax.axis_index("c")
        s = lax.axis_index("s")
        wid = s * nc + c

        zvec = jnp.zeros((16,), jnp.float32)
        stores_per_row = d // 16

        def zbody(i, _):
            r = i // stores_per_row
            col = (i % stores_per_row) * 16
            zero_v[r, pl.ds(col, 16)] = zvec
            return 0

        lax.fori_loop(0, zblk * stores_per_row, zbody, 0)
        nz = rows_blk // zblk
        for b in range(nz):
            pltpu.async_copy(zero_v,
                             acc_sh.at[pl.ds(s * rows_blk + b * zblk, zblk)],
                             zsem)
        if tail:
            @pl.when(s == 0)
            def _():
                pltpu.sync_copy(zero_v.at[pl.ds(0, tail)],
                                acc_sh.at[pl.ds(ns * rows_blk, tail)])

        def start_gather(sl, i, b):
            pltpu.async_copy(x_hbm.at[srcs[sl].at[i]], bufs[b], gsem[b])

        def wait_gather(b):
            pltpu.make_async_copy(x_hbm.at[pl.ds(0, chunk)], bufs[b],
                                  gsem[b]).wait()

        def start_scatter(sl, i, b):
            pltpu.async_copy(bufs[b], acc_sh.at[dsts[sl].at[i]], ssem[b],
                             add=True)

        def wait_scatter(b):
            pltpu.make_async_copy(bufs[b], acc_sh.at[pl.ds(0, chunk)],
                                  ssem[b]).wait()

        def stage_idx(sl, g):
            pltpu.sync_copy(e_hbm.at[0, wid, g], srcs[sl])
            pltpu.sync_copy(e_hbm.at[1, wid, g], dsts[sl])

        stage_idx(0, 0)
        for b in range(nbuf):
            start_gather(0, b, b)
        for b in range(nz):
            pltpu.make_async_copy(
                zero_v, acc_sh.at[pl.ds(s * rows_blk + b * zblk, zblk)],
                zsem).wait()
        plsc.subcore_barrier()

        for g in range(n_grps):
            sl = g % 2

            def round_(r, _, sl=sl):
                i0 = r * nbuf
                for b in range(nbuf):
                    wait_gather(b)
                    start_scatter(sl, i0 + b, b)
                for b in range(nbuf):
                    wait_scatter(b)
                    start_gather(sl, i0 + nbuf + b, b)
                return 0

            lax.fori_loop(0, n_rounds - 1, round_, 0)
            i0 = (n_rounds - 1) * nbuf
            for b in range(nbuf):
                wait_gather(b)
                start_scatter(sl, i0 + b, b)
            if g + 1 < n_grps:
                stage_idx(1 - sl, g + 1)
                for b in range(nbuf):
                    wait_scatter(b)
                    start_gather(1 - sl, b, b)
            else:
                for b in range(nbuf):
                    wait_scatter(b)
        plsc.subcore_barrier()

        pltpu.sync_copy(
            acc_sh.at[pl.ds(s * rows_blk, rows_blk)],
            out_hbm.at[c, pl.ds(s * rows_blk, rows_blk)],
        )
        if tail:
            @pl.when(s == 0)
            def _():
                pltpu.sync_copy(
                    acc_sh.at[pl.ds(ns * rows_blk, tail)],
                    out_hbm.at[c, pl.ds(ns * rows_blk, tail)],
                )

    e5 = edge_index.reshape(2, nw, n_grps, grp, chunk)
    return agg_kernel(x, e5)


_row_spec = lambda shape: pl.BlockSpec(shape, lambda i: (i, 0))
_full_spec = lambda shape: pl.BlockSpec(shape, lambda i: (0, 0))


def _tc_pre(x, W_root, b_rel, blk):
    n, d_in = x.shape

    def body(xr, wroot, brel, out):
        out[...] = (jnp.dot(xr[...], wroot[...],
                            preferred_element_type=jnp.float32) + brel[...])

    return pl.pallas_call(
        body,
        grid=(n // blk,),
        in_specs=[
            _row_spec((blk, d_in)),
            _full_spec((d_in, d_in)),
            _full_spec((1, d_in)),
        ],
        out_specs=_row_spec((blk, d_in)),
        out_shape=jax.ShapeDtypeStruct((n, d_in), jnp.float32),
    )(x, W_root, b_rel.reshape(1, -1))


def _tc_post(parts, pre, W_rel, W1, b1, W2, b2, blk):
    n, d_in = pre.shape
    d_hid = W1.shape[1]
    d_out = W2.shape[1]

    def body(pa, pr, wrel, w1, b1r, w2, b2r, out):
        agg = pa[0] + pa[1]
        h = jnp.dot(agg, wrel[...], preferred_element_type=jnp.float32)
        h += pr[...]
        h2 = jnp.dot(h, w1[...], preferred_element_type=jnp.float32) + b1r[...]
        h2 = jnp.maximum(h2, 0.0)
        o = jnp.dot(h2, w2[...], preferred_element_type=jnp.float32) + b2r[...]
        out[...] = jax.nn.sigmoid(o)

    return pl.pallas_call(
        body,
        grid=(n // blk,),
        in_specs=[
            pl.BlockSpec((2, blk, d_in), lambda i: (0, i, 0)),
            _row_spec((blk, d_in)),
            _full_spec((d_in, d_in)),
            _full_spec((d_in, d_hid)),
            _full_spec((1, d_hid)),
            _full_spec((d_hid, d_out)),
            _full_spec((1, d_out)),
        ],
        out_specs=_row_spec((blk, d_out)),
        out_shape=jax.ShapeDtypeStruct((n, d_out), jnp.float32),
    )(parts, pre, W_rel, W1, b1.reshape(1, -1), W2, b2.reshape(1, -1))


@jax.jit
def kernel(x, edge_index, W_rel, b_rel, W_root, W1, b1, W2, b2):
    n, d = x.shape
    e = edge_index.shape[1]
    ei = edge_index if edge_index.dtype == jnp.int32 else edge_index.astype(jnp.int32)
    parts = _sc_aggregate(x, ei, n, e, d)
    pre = _tc_pre(x, W_root, b_rel, blk=2000)
    return _tc_post(parts, pre, W_rel, W1, b1, W2, b2, blk=2000)


# speedup vs baseline: 1.0348x; 1.0348x over previous
import functools

import jax
import jax.numpy as jnp
from jax import lax
from jax.experimental import pallas as pl
from jax.experimental.pallas import tpu as pltpu
from jax.experimental.pallas import tpu_sc as plsc


def _sc_aggregate(x, edge_index, n_nodes, n_edges, d):
    info = plsc.get_sparse_core_info()
    nc, ns = info.num_cores, info.num_subcores
    nw = nc * ns

    epw = n_edges // nw
    chunk = 40
    n_chunks = epw // chunk
    nbuf = 5
    grp = 50
    n_grps = n_chunks // grp
    n_rounds = grp // nbuf
    assert epw % chunk == 0 and n_edges % nw == 0
    assert n_chunks % grp == 0 and grp % nbuf == 0 and n_rounds >= 2
    rows_blk = (n_nodes // ns) // 8 * 8
    tail = n_nodes - ns * rows_blk
    zblk = 16
    assert rows_blk % zblk == 0 and zblk % 8 == 0 and tail % 8 == 0
    assert tail <= zblk

    mesh = plsc.VectorSubcoreMesh(core_axis_name="c", subcore_axis_name="s")

    @functools.partial(
        pl.kernel,
        mesh=mesh,
        out_type=jax.ShapeDtypeStruct((nc, n_nodes, d), jnp.float32),
        scratch_types=(
            [
                pltpu.VMEM((grp, chunk), jnp.int32),
                pltpu.VMEM((grp, chunk), jnp.int32),
            ]
            + [pltpu.VMEM((chunk, d), jnp.float32) for _ in range(nbuf)]
            + [
                pltpu.VMEM_SHARED((n_nodes, d), jnp.float32),
                pltpu.VMEM((zblk, d), jnp.float32),
            ]
            + [pltpu.SemaphoreType.DMA for _ in range(2 * nbuf + 1)]
        ),
    )
    def agg_kernel(x_hbm, e_hbm, out_hbm, src_v, dst_v, *rest):
        bufs = rest[:nbuf]
        acc_sh = rest[nbuf]
        zero_v = rest[nbuf + 1]
        gsem = rest[nbuf + 2:2 * nbuf + 2]
        ssem = rest[2 * nbuf + 2:3 * nbuf + 2]
        zsem = rest[3 * nbuf + 2]
        c = lax.axis_index("c")
        s = lax.axis_index("s")
        wid = s * nc + c

        zvec = jnp.zeros((16,), jnp.float32)
        stores_per_row = d // 16

        def zbody(i, _):
            r = i // stores_per_row
            col = (i % stores_per_row) * 16
            zero_v[r, pl.ds(col, 16)] = zvec
            return 0

        lax.fori_loop(0, zblk * stores_per_row, zbody, 0)
        nz = rows_blk // zblk
        for b in range(nz):
            pltpu.async_copy(zero_v,
                             acc_sh.at[pl.ds(s * rows_blk + b * zblk, zblk)],
                             zsem)
        if tail:
            @pl.when(s == 0)
            def _():
                pltpu.sync_copy(zero_v.at[pl.ds(0, tail)],
                                acc_sh.at[pl.ds(ns * rows_blk, tail)])

        def start_gather(i, b):
            pltpu.async_copy(x_hbm.at[src_v.at[i]], bufs[b], gsem[b])

        def wait_gather(b):
            pltpu.make_async_copy(x_hbm.at[pl.ds(0, chunk)], bufs[b],
                                  gsem[b]).wait()

        def start_scatter(i, b):
            pltpu.async_copy(bufs[b], acc_sh.at[dst_v.at[i]], ssem[b],
                             add=True)

        def wait_scatter(b):
            pltpu.make_async_copy(bufs[b], acc_sh.at[pl.ds(0, chunk)],
                                  ssem[b]).wait()

        for b in range(nz):
            pltpu.make_async_copy(
                zero_v, acc_sh.at[pl.ds(s * rows_blk + b * zblk, zblk)],
                zsem).wait()
        plsc.subcore_barrier()

        def group(g, _):
            pltpu.sync_copy(e_hbm.at[0, wid, g], src_v)
            pltpu.sync_copy(e_hbm.at[1, wid, g], dst_v)
            for b in range(nbuf):
                start_gather(b, b)

            def round_(r, _):
                i0 = r * nbuf
                for b in range(nbuf):
                    wait_gather(b)
                    start_scatter(i0 + b, b)
                for b in range(nbuf):
                    wait_scatter(b)
                    start_gather(i0 + nbuf + b, b)
                return 0

            lax.fori_loop(0, n_rounds - 1, round_, 0)
            i0 = (n_rounds - 1) * nbuf
            for b in range(nbuf):
                wait_gather(b)
                start_scatter(i0 + b, b)
            for b in range(nbuf):
                wait_scatter(b)
            return 0

        lax.fori_loop(0, n_grps, group, 0)
        plsc.subcore_barrier()

        pltpu.sync_copy(
            acc_sh.at[pl.ds(s * rows_blk, rows_blk)],
            out_hbm.at[c, pl.ds(s * rows_blk, rows_blk)],
        )
        if tail:
            @pl.when(s == 0)
            def _():
                pltpu.sync_copy(
                    acc_sh.at[pl.ds(ns * rows_blk, tail)],
                    out_hbm.at[c, pl.ds(ns * rows_blk, tail)],
                )

    e5 = edge_index.reshape(2, nw, n_grps, grp, chunk)
    return agg_kernel(x, e5)


_row_spec = lambda shape: pl.BlockSpec(shape, lambda i: (i, 0))
_full_spec = lambda shape: pl.BlockSpec(shape, lambda i: (0, 0))


def _tc_pre(x, W_root, b_rel, blk):
    n, d_in = x.shape

    def body(xr, wroot, brel, out):
        out[...] = (jnp.dot(xr[...], wroot[...],
                            preferred_element_type=jnp.float32) + brel[...])

    return pl.pallas_call(
        body,
        grid=(n // blk,),
        in_specs=[
            _row_spec((blk, d_in)),
            _full_spec((d_in, d_in)),
            _full_spec((1, d_in)),
        ],
        out_specs=_row_spec((blk, d_in)),
        out_shape=jax.ShapeDtypeStruct((n, d_in), jnp.float32),
    )(x, W_root, b_rel.reshape(1, -1))


def _tc_post(parts, pre, W_rel, W1, b1, W2, b2, blk):
    n, d_in = pre.shape
    d_hid = W1.shape[1]
    d_out = W2.shape[1]

    def body(pa, pr, wrel, w1, b1r, w2, b2r, out):
        agg = pa[0] + pa[1]
        h = jnp.dot(agg, wrel[...], preferred_element_type=jnp.float32)
        h += pr[...]
        h2 = jnp.dot(h, w1[...], preferred_element_type=jnp.float32) + b1r[...]
        h2 = jnp.maximum(h2, 0.0)
        o = jnp.dot(h2, w2[...], preferred_element_type=jnp.float32) + b2r[...]
        out[...] = jax.nn.sigmoid(o)

    return pl.pallas_call(
        body,
        grid=(n // blk,),
        in_specs=[
            pl.BlockSpec((2, blk, d_in), lambda i: (0, i, 0)),
            _row_spec((blk, d_in)),
            _full_spec((d_in, d_in)),
            _full_spec((d_in, d_hid)),
            _full_spec((1, d_hid)),
            _full_spec((d_hid, d_out)),
            _full_spec((1, d_out)),
        ],
        out_specs=_row_spec((blk, d_out)),
        out_shape=jax.ShapeDtypeStruct((n, d_out), jnp.float32),
    )(parts, pre, W_rel, W1, b1.reshape(1, -1), W2, b2.reshape(1, -1))


@jax.jit
def kernel(x, edge_index, W_rel, b_rel, W_root, W1, b1, W2, b2):
    n, d = x.shape
    e = edge_index.shape[1]
    ei = edge_index if edge_index.dtype == jnp.int32 else edge_index.astype(jnp.int32)
    parts = _sc_aggregate(x, ei, n, e, d)
    pre = _tc_pre(x, W_root, b_rel, blk=2000)
    return _tc_post(parts, pre, W_rel, W1, b1, W2, b2, blk=2000)
